# baseline (speedup 1.0000x reference)
import functools
import math

import jax
import jax.numpy as jnp
from jax import lax
from jax.experimental import pallas as pl
from jax.experimental.pallas import tpu as pltpu
from jax.experimental.pallas import tpu_sc as plsc

D = 1024
H = 2048
E = 8
T = 2048
CAP = math.floor(T * 0.5)

P = 256
NT = (T + E * (P - 1)) // P
HT = 2048
NH = H // HT
CH = 256
NCH = T // CH


def _router_body(x_ref, w_ref, b_ref, d_ref, wgt_ref, te_ref,
                 oh1_ref, oh2_ref, ohm_ref, c1_ref, c2_ref, cm_ref):
    x = x_ref[...]
    scores = jnp.dot(x, w_ref[...],
                     preferred_element_type=jnp.float32) + b_ref[...]
    lane_e = lax.broadcasted_iota(jnp.int32, (T, E), 1)
    v1 = jnp.max(scores, axis=1, keepdims=True)
    i1 = jnp.min(jnp.where(scores == v1, lane_e, E), axis=1, keepdims=True)
    masked = jnp.where(lane_e == i1, -jnp.inf, scores)
    v2 = jnp.max(masked, axis=1, keepdims=True)
    i2 = jnp.min(jnp.where(masked == v2, lane_e, E), axis=1, keepdims=True)
    s2 = 1.0 / (1.0 + jnp.exp(v1 - v2))
    s1 = 1.0 - s2
    oh1_ref[...] = (lane_e == i1).astype(jnp.float32)
    oh2_ref[...] = (lane_e == i2).astype(jnp.float32)

    tri = (lax.broadcasted_iota(jnp.int32, (CH, CH), 0) >=
           lax.broadcasted_iota(jnp.int32, (CH, CH), 1)).astype(jnp.float32)

    def passA(k, carry):
        ca, cb = carry
        o1 = oh1_ref[pl.ds(k * CH, CH), :]
        o2 = oh2_ref[pl.ds(k * CH, CH), :]
        cc1 = jnp.dot(tri, o1, preferred_element_type=jnp.float32) + ca
        cc2 = jnp.dot(tri, o2, preferred_element_type=jnp.float32) + cb
        c1_ref[pl.ds(k * CH, CH), :] = cc1
        c2_ref[pl.ds(k * CH, CH), :] = cc2
        return (cc1[CH - 1:CH, :], cc2[CH - 1:CH, :])

    zero8 = jnp.zeros((1, E), jnp.float32)
    lax.fori_loop(0, NCH, passA, (zero8, zero8))

    oh1 = oh1_ref[...]
    oh2 = oh2_ref[...]
    c1 = c1_ref[...]
    c12 = c1 + c2_ref[...]
    pos1 = jnp.sum(oh1 * c1, axis=1, keepdims=True)
    pos2 = jnp.sum(oh2 * c12, axis=1, keepdims=True)
    keep1 = pos1 < CAP
    keep2 = pos2 < CAP
    m = jnp.maximum(jnp.where(keep1, i1, 0), jnp.where(keep2, i2, 0))
    wgt_ref[...] = jnp.where(keep1, s1, 0.0) + jnp.where(keep2, s2, 0.0)
    ohm_ref[...] = (lane_e == m).astype(jnp.float32)

    def passB(k, carry):
        om = ohm_ref[pl.ds(k * CH, CH), :]
        cc = jnp.dot(tri, om, preferred_element_type=jnp.float32) + carry
        cm_ref[pl.ds(k * CH, CH), :] = cc
        return cc[CH - 1:CH, :]

    counts = lax.fori_loop(0, NCH, passB, zero8)
    ohm = ohm_ref[...]
    rank = jnp.sum(ohm * cm_ref[...], axis=1, keepdims=True) - 1.0

    tiles = jnp.floor((counts + (P - 1)) / P)
    tri8 = (lax.broadcasted_iota(jnp.int32, (E, E), 0) <
            lax.broadcasted_iota(jnp.int32, (E, E), 1)).astype(jnp.float32)
    tstart = jnp.dot(tiles, tri8, preferred_element_type=jnp.float32)
    row_start = tstart * P
    d = jnp.sum(ohm * row_start, axis=1, keepdims=True) + rank
    d_ref[...] = d.astype(jnp.int32)

    tend = tstart + tiles
    i_nt = lax.broadcasted_iota(jnp.int32, (NT, E), 0).astype(jnp.float32)
    cnt = jnp.sum(jnp.where(tend <= i_nt, 1.0, 0.0), axis=1, keepdims=True)
    te_ref[0:NT, :] = jnp.minimum(cnt, E - 1).astype(jnp.int32)
    te_ref[NT:NT + 1, :] = jnp.sum(tiles, axis=1, keepdims=True).astype(jnp.int32)


def _router_call(x2, router_w, router_b2):
    return pl.pallas_call(
        _router_body,
        out_shape=(
            jax.ShapeDtypeStruct((T, 1), jnp.int32),
            jax.ShapeDtypeStruct((T, 1), jnp.float32),
            jax.ShapeDtypeStruct((NT + 1, 1), jnp.int32),
        ),
        scratch_shapes=[pltpu.VMEM((T, E), jnp.float32)] * 6,
        interpret=False,
    )(x2, router_w, router_b2)


_NW = 32
_TPW = T // _NW


def _sc_mesh():
    return plsc.VectorSubcoreMesh(core_axis_name="c", subcore_axis_name="s")


def _dispatch_sc(x_hbm, d_hbm, xs_hbm, idx_v, rows_v, sem):
    wid = lax.axis_index("s") * 2 + lax.axis_index("c")
    base = wid * _TPW
    pltpu.sync_copy(d_hbm.at[pl.ds(base, _TPW)], idx_v)
    pltpu.sync_copy(x_hbm.at[pl.ds(base, _TPW)], rows_v)
    pltpu.async_copy(rows_v, xs_hbm.at[idx_v], sem).wait()


def _dispatch_call(x2, d):
    k = pl.kernel(
        _dispatch_sc,
        out_type=jax.ShapeDtypeStruct((NT * P, D), jnp.float32),
        mesh=_sc_mesh(),
        scratch_types=[
            pltpu.VMEM((_TPW,), jnp.int32),
            pltpu.VMEM((_TPW, D), jnp.float32),
            pltpu.SemaphoreType.DMA,
        ],
    )
    return k(x2, d)


def _gather_sc(tpw, tab_hbm, d_hbm, y_hbm, idx_v, rows_v, sem):
    wid = lax.axis_index("s") * 2 + lax.axis_index("c")
    base = wid * tpw
    pltpu.sync_copy(d_hbm.at[pl.ds(base, tpw)], idx_v)
    pltpu.async_copy(tab_hbm.at[idx_v], rows_v, sem).wait()
    pltpu.sync_copy(rows_v, y_hbm.at[pl.ds(base, tpw)])


def _gather_call(tab, d, n):
    tpw = n // _NW
    k = pl.kernel(
        functools.partial(_gather_sc, tpw),
        out_type=jax.ShapeDtypeStruct((n, D), jnp.float32),
        mesh=_sc_mesh(),
        scratch_types=[
            pltpu.VMEM((tpw,), jnp.int32),
            pltpu.VMEM((tpw, D), jnp.float32),
            pltpu.SemaphoreType.DMA,
        ],
    )
    return k(tab, d)


def _ffn_body(te_ref, xs_ref, w1_ref, w2_ref, w3_ref, out_ref):
    i_idx = pl.program_id(0)
    h_idx = pl.program_id(1)

    @pl.when(i_idx < te_ref[NT])
    def _():
        @pl.when(h_idx == 0)
        def _():
            out_ref[...] = jnp.zeros_like(out_ref)

        xt = xs_ref[...]
        a = jnp.dot(xt, w2_ref[0], preferred_element_type=jnp.float32)
        g = jnp.dot(xt, w1_ref[0], preferred_element_type=jnp.float32)
        h = a * g
        h = 0.5 * h * (1.0 + lax.erf(h * (1.0 / math.sqrt(2.0))))
        out_ref[...] += jnp.dot(h, w3_ref[0], preferred_element_type=jnp.float32)


def _ffn_call(xs, te, W1, W2, W3):
    def _tile(i, s):
        return jnp.where(i < s[NT], i, s[NT] - 1)

    grid_spec = pltpu.PrefetchScalarGridSpec(
        num_scalar_prefetch=1,
        grid=(NT, NH),
        in_specs=[
            pl.BlockSpec((P, D), lambda i, h, s: (_tile(i, s), 0)),
            pl.BlockSpec((1, D, HT),
                         lambda i, h, s: (s[_tile(i, s)], 0,
                                          jnp.where(i < s[NT], h, NH - 1))),
            pl.BlockSpec((1, D, HT),
                         lambda i, h, s: (s[_tile(i, s)], 0,
                                          jnp.where(i < s[NT], h, NH - 1))),
            pl.BlockSpec((1, HT, D),
                         lambda i, h, s: (s[_tile(i, s)],
                                          jnp.where(i < s[NT], h, NH - 1), 0)),
        ],
        out_specs=pl.BlockSpec((P, D), lambda i, h, s: (_tile(i, s), 0)),
    )
    return pl.pallas_call(
        _ffn_body,
        grid_spec=grid_spec,
        out_shape=jax.ShapeDtypeStruct((NT * P, D), jnp.float32),
        interpret=False,
    )(te, xs, W1, W2, W3)


def _combine_body(x_ref, y_ref, w_ref, o_ref):
    w = w_ref[...]
    o_ref[...] = w * y_ref[...] + (1.0 - w) * x_ref[...]


def _combine_call(x2, y, wgt, n):
    return pl.pallas_call(
        _combine_body,
        grid=(n // P,),
        in_specs=[
            pl.BlockSpec((P, D), lambda i: (i, 0)),
            pl.BlockSpec((P, D), lambda i: (i, 0)),
            pl.BlockSpec((P, 1), lambda i: (i, 0)),
        ],
        out_specs=pl.BlockSpec((P, D), lambda i: (i, 0)),
        out_shape=jax.ShapeDtypeStruct((n, D), jnp.float32),
        interpret=False,
    )(x2, y, wgt)


def kernel(x, router_w, router_b, W1, W2, W3):
    b, t, _ = x.shape
    x2 = x.reshape(T, D)
    d2, wgt, te = _router_call(x2, router_w, router_b.reshape(1, E))
    d = d2.reshape(T)
    xs = _dispatch_call(x2, d)
    out_buf = _ffn_call(xs, te.reshape(NT + 1), W1, W2, W3)
    T2 = T // 2
    y0 = _gather_call(out_buf, d[:T2], T2)
    y1 = _gather_call(out_buf, d[T2:], T2)
    r0 = _combine_call(x2[:T2], y0, wgt[:T2], T2)
    r1 = _combine_call(x2[T2:], y1, wgt[T2:], T2)
    res = jnp.concatenate([r0, r1], axis=0)
    return res.reshape(b, t, D)


# speedup vs baseline: 1.1181x; 1.1181x over previous
import functools
import math

import jax
import jax.numpy as jnp
from jax import lax
from jax.experimental import pallas as pl
from jax.experimental.pallas import tpu as pltpu
from jax.experimental.pallas import tpu_sc as plsc

D = 1024
H = 2048
E = 8
T = 2048
CAP = math.floor(T * 0.5)

P = 256
NT = (T + E * (P - 1)) // P
HT = 2048
NH = H // HT
CH = 256
NCH = T // CH


def _router_body(x_ref, w_ref, b_ref, d_ref, wgt_ref, te_ref,
                 oh1_ref, oh2_ref, ohm_ref, c1_ref, c2_ref, cm_ref):
    x = x_ref[...]
    scores = jnp.dot(x, w_ref[...],
                     preferred_element_type=jnp.float32) + b_ref[...]
    lane_e = lax.broadcasted_iota(jnp.int32, (T, E), 1)
    v1 = jnp.max(scores, axis=1, keepdims=True)
    i1 = jnp.min(jnp.where(scores == v1, lane_e, E), axis=1, keepdims=True)
    masked = jnp.where(lane_e == i1, -jnp.inf, scores)
    v2 = jnp.max(masked, axis=1, keepdims=True)
    i2 = jnp.min(jnp.where(masked == v2, lane_e, E), axis=1, keepdims=True)
    s2 = 1.0 / (1.0 + jnp.exp(v1 - v2))
    s1 = 1.0 - s2
    oh1_ref[...] = (lane_e == i1).astype(jnp.float32)
    oh2_ref[...] = (lane_e == i2).astype(jnp.float32)

    tri = (lax.broadcasted_iota(jnp.int32, (CH, CH), 0) >=
           lax.broadcasted_iota(jnp.int32, (CH, CH), 1)).astype(jnp.float32)

    def passA(k, carry):
        ca, cb = carry
        o1 = oh1_ref[pl.ds(k * CH, CH), :]
        o2 = oh2_ref[pl.ds(k * CH, CH), :]
        cc1 = jnp.dot(tri, o1, preferred_element_type=jnp.float32) + ca
        cc2 = jnp.dot(tri, o2, preferred_element_type=jnp.float32) + cb
        c1_ref[pl.ds(k * CH, CH), :] = cc1
        c2_ref[pl.ds(k * CH, CH), :] = cc2
        return (cc1[CH - 1:CH, :], cc2[CH - 1:CH, :])

    zero8 = jnp.zeros((1, E), jnp.float32)
    lax.fori_loop(0, NCH, passA, (zero8, zero8))

    oh1 = oh1_ref[...]
    oh2 = oh2_ref[...]
    c1 = c1_ref[...]
    c12 = c1 + c2_ref[...]
    pos1 = jnp.sum(oh1 * c1, axis=1, keepdims=True)
    pos2 = jnp.sum(oh2 * c12, axis=1, keepdims=True)
    keep1 = pos1 < CAP
    keep2 = pos2 < CAP
    m = jnp.maximum(jnp.where(keep1, i1, 0), jnp.where(keep2, i2, 0))
    wgt_ref[...] = jnp.where(keep1, s1, 0.0) + jnp.where(keep2, s2, 0.0)
    ohm_ref[...] = (lane_e == m).astype(jnp.float32)

    def passB(k, carry):
        om = ohm_ref[pl.ds(k * CH, CH), :]
        cc = jnp.dot(tri, om, preferred_element_type=jnp.float32) + carry
        cm_ref[pl.ds(k * CH, CH), :] = cc
        return cc[CH - 1:CH, :]

    counts = lax.fori_loop(0, NCH, passB, zero8)
    ohm = ohm_ref[...]
    rank = jnp.sum(ohm * cm_ref[...], axis=1, keepdims=True) - 1.0

    tiles = jnp.floor((counts + (P - 1)) / P)
    tri8 = (lax.broadcasted_iota(jnp.int32, (E, E), 0) <
            lax.broadcasted_iota(jnp.int32, (E, E), 1)).astype(jnp.float32)
    tstart = jnp.dot(tiles, tri8, preferred_element_type=jnp.float32)
    row_start = tstart * P
    d = jnp.sum(ohm * row_start, axis=1, keepdims=True) + rank
    d_ref[...] = d.astype(jnp.int32)

    tend = tstart + tiles
    i_nt = lax.broadcasted_iota(jnp.int32, (NT, E), 0).astype(jnp.float32)
    cnt = jnp.sum(jnp.where(tend <= i_nt, 1.0, 0.0), axis=1, keepdims=True)
    te_ref[0:NT, :] = jnp.minimum(cnt, E - 1).astype(jnp.int32)
    te_ref[NT:NT + 1, :] = jnp.sum(tiles, axis=1, keepdims=True).astype(jnp.int32)


def _router_call(x2, router_w, router_b2):
    return pl.pallas_call(
        _router_body,
        out_shape=(
            jax.ShapeDtypeStruct((T, 1), jnp.int32),
            jax.ShapeDtypeStruct((T, 1), jnp.float32),
            jax.ShapeDtypeStruct((NT + 1, 1), jnp.int32),
        ),
        scratch_shapes=[pltpu.VMEM((T, E), jnp.float32)] * 6,
        interpret=False,
    )(x2, router_w, router_b2)


_NW = 32
_TPW = T // _NW


def _sc_mesh():
    return plsc.VectorSubcoreMesh(core_axis_name="c", subcore_axis_name="s")


def _dispatch_sc(x_hbm, d_hbm, xs_hbm, idx_v, rows_v, sem):
    wid = lax.axis_index("s") * 2 + lax.axis_index("c")
    base = wid * _TPW
    pltpu.sync_copy(d_hbm.at[pl.ds(base, _TPW)], idx_v)
    pltpu.sync_copy(x_hbm.at[pl.ds(base, _TPW)], rows_v)
    pltpu.async_copy(rows_v, xs_hbm.at[idx_v], sem).wait()


def _dispatch_call(x2, d):
    k = pl.kernel(
        _dispatch_sc,
        out_type=jax.ShapeDtypeStruct((NT * P, D), jnp.float32),
        mesh=_sc_mesh(),
        scratch_types=[
            pltpu.VMEM((_TPW,), jnp.int32),
            pltpu.VMEM((_TPW, D), jnp.float32),
            pltpu.SemaphoreType.DMA,
        ],
    )
    return k(x2, d)


def _gather_sc(tpw, tab_hbm, d_hbm, y_hbm, idx_v, rows_v, sem):
    wid = lax.axis_index("s") * 2 + lax.axis_index("c")
    base = wid * tpw
    pltpu.sync_copy(d_hbm.at[pl.ds(base, tpw)], idx_v)
    pltpu.async_copy(tab_hbm.at[idx_v], rows_v, sem).wait()
    pltpu.sync_copy(rows_v, y_hbm.at[pl.ds(base, tpw)])


def _gather_call(tab, d, n):
    tpw = n // _NW
    k = pl.kernel(
        functools.partial(_gather_sc, tpw),
        out_type=jax.ShapeDtypeStruct((n, D), jnp.float32),
        mesh=_sc_mesh(),
        scratch_types=[
            pltpu.VMEM((tpw,), jnp.int32),
            pltpu.VMEM((tpw, D), jnp.float32),
            pltpu.SemaphoreType.DMA,
        ],
    )
    return k(tab, d)


def _ffn_body(te_ref, xs_ref, w1_ref, w2_ref, w3_ref, out_ref):
    i_idx = pl.program_id(0)
    h_idx = pl.program_id(1)

    @pl.when(i_idx < te_ref[NT])
    def _():
        @pl.when(h_idx == 0)
        def _():
            out_ref[...] = jnp.zeros_like(out_ref)

        xt = xs_ref[...]
        a = jnp.dot(xt, w2_ref[0], preferred_element_type=jnp.float32)
        g = jnp.dot(xt, w1_ref[0], preferred_element_type=jnp.float32)
        h = a * g
        h = 0.5 * h * (1.0 + lax.erf(h * (1.0 / math.sqrt(2.0))))
        out_ref[...] += jnp.dot(h, w3_ref[0], preferred_element_type=jnp.float32)


def _ffn_call(xs, te, W1, W2, W3):
    def _tile(i, s):
        return jnp.where(i < s[NT], i, s[NT] - 1)

    grid_spec = pltpu.PrefetchScalarGridSpec(
        num_scalar_prefetch=1,
        grid=(NT, NH),
        in_specs=[
            pl.BlockSpec((P, D), lambda i, h, s: (_tile(i, s), 0)),
            pl.BlockSpec((1, D, HT),
                         lambda i, h, s: (s[_tile(i, s)], 0,
                                          jnp.where(i < s[NT], h, NH - 1))),
            pl.BlockSpec((1, D, HT),
                         lambda i, h, s: (s[_tile(i, s)], 0,
                                          jnp.where(i < s[NT], h, NH - 1))),
            pl.BlockSpec((1, HT, D),
                         lambda i, h, s: (s[_tile(i, s)],
                                          jnp.where(i < s[NT], h, NH - 1), 0)),
        ],
        out_specs=pl.BlockSpec((P, D), lambda i, h, s: (_tile(i, s), 0)),
    )
    return pl.pallas_call(
        _ffn_body,
        grid_spec=grid_spec,
        out_shape=jax.ShapeDtypeStruct((NT * P, D), jnp.float32),
        interpret=False,
    )(te, xs, W1, W2, W3)


def _combine_body(x_ref, y_ref, w_ref, o_ref):
    w = w_ref[...]
    o_ref[...] = w * y_ref[...] + (1.0 - w) * x_ref[...]


def _combine_call(x2, y, wgt, n):
    return pl.pallas_call(
        _combine_body,
        grid=(n // P,),
        in_specs=[
            pl.BlockSpec((P, D), lambda i: (i, 0)),
            pl.BlockSpec((P, D), lambda i: (i, 0)),
            pl.BlockSpec((P, 1), lambda i: (i, 0)),
        ],
        out_specs=pl.BlockSpec((P, D), lambda i: (i, 0)),
        out_shape=jax.ShapeDtypeStruct((n, D), jnp.float32),
        interpret=False,
    )(x2, y, wgt)


def kernel(x, router_w, router_b, W1, W2, W3):
    b, t, _ = x.shape
    x2 = x.reshape(T, D)
    d2, wgt, te = _router_call(x2, router_w, router_b.reshape(1, E))
    d = d2.reshape(T)
    xs = _dispatch_call(x2, d)
    out_buf = _ffn_call(xs, te.reshape(NT + 1), W1, W2, W3)
    y = _gather_call(out_buf, d, T)
    res = _combine_call(x2, y, wgt, T)
    return res.reshape(b, t, D)
